# Initial kernel scaffold; baseline (speedup 1.0000x reference)
import jax
import jax.numpy as jnp
from jax.experimental import pallas as pl


def kernel(pos_h, pos_t, pos_r, neg_h, neg_t, neg_r, ent_embeddings, rel_embeddings, ent_transfer, rel_transfer):
    raise NotImplementedError("write your pallas kernel here")


# speedup vs baseline: 1.7756x; 1.7756x over previous
import functools

import jax
import jax.numpy as jnp
from jax import lax
from jax.experimental import pallas as pl
from jax.experimental.pallas import tpu as pltpu
from jax.experimental.pallas import tpu_sc as plsc

BATCH = 16384
DIM = 128
NUM_WORKERS = 32
ROWS_PER_WORKER = BATCH // NUM_WORKERS
CHUNK = 128
NUM_CHUNKS = ROWS_PER_WORKER // CHUNK
MARGIN = 1.0

_ROW_TYPE = jax.ShapeDtypeStruct((BATCH, DIM), jnp.float32)


def _make_gather_kernel():
    mesh = plsc.VectorSubcoreMesh(core_axis_name="c", subcore_axis_name="s")

    @functools.partial(
        pl.kernel,
        out_type=[_ROW_TYPE] * 12,
        mesh=mesh,
        scratch_types=[
            pltpu.VMEM((CHUNK,), jnp.int32),
            pltpu.VMEM((CHUNK, DIM), jnp.float32),
            pltpu.SemaphoreType.DMA,
        ],
    )
    def gather12(ent_e, ent_t, rel_e, rel_t,
                 ph, pt, pr, nh, nt, nr,
                 o_phe, o_pte, o_php, o_ptp, o_pre, o_prp,
                 o_nhe, o_nte, o_nhp, o_ntp, o_nre, o_nrp,
                 idx_v, rows_v, sem):
        wid = lax.axis_index("s") * 2 + lax.axis_index("c")
        base0 = wid * ROWS_PER_WORKER
        combos = [
            (ent_e, ph, o_phe), (ent_e, pt, o_pte),
            (ent_t, ph, o_php), (ent_t, pt, o_ptp),
            (rel_e, pr, o_pre), (rel_t, pr, o_prp),
            (ent_e, nh, o_nhe), (ent_e, nt, o_nte),
            (ent_t, nh, o_nhp), (ent_t, nt, o_ntp),
            (rel_e, nr, o_nre), (rel_t, nr, o_nrp),
        ]
        for tbl_, idx_, out_ in combos:
            @pl.loop(0, NUM_CHUNKS)
            def _(c, tbl=tbl_, idx=idx_, out=out_):
                b = base0 + c * CHUNK
                pltpu.sync_copy(idx.at[pl.ds(b, CHUNK)], idx_v)
                pltpu.async_copy(tbl.at[idx_v], rows_v, sem).wait()
                pltpu.sync_copy(rows_v, out.at[pl.ds(b, CHUNK)])

    return gather12


_gather12 = _make_gather_kernel()

_TC_ROWS = 1024
_TC_GRID = BATCH // _TC_ROWS


def _loss_body(phe, pte, php, ptp, pre, prp,
               nhe, nte, nhp, ntp, nre, nrp, out_ref):
    i = pl.program_id(0)
    he = phe[...]
    te = pte[...]
    sp = jnp.sum(he * php[...] - te * ptp[...], axis=1, keepdims=True)
    dp = he - te + pre[...] + sp * prp[...]
    ps = jnp.sum(dp * dp, axis=1)

    he_n = nhe[...]
    te_n = nte[...]
    sn = jnp.sum(he_n * nhp[...] - te_n * ntp[...], axis=1, keepdims=True)
    dn = he_n - te_n + nre[...] + sn * nrp[...]
    ns = jnp.sum(dn * dn, axis=1)

    partial = jnp.sum(jnp.maximum(ns - ps + MARGIN, 0.0))

    @pl.when(i == 0)
    def _():
        out_ref[0, 0] = 0.0

    out_ref[0, 0] += partial


_row_spec = pl.BlockSpec((_TC_ROWS, DIM), lambda i: (i, 0))

_loss_call = pl.pallas_call(
    _loss_body,
    grid=(_TC_GRID,),
    in_specs=[_row_spec] * 12,
    out_specs=pl.BlockSpec((1, 1), lambda i: (0, 0), memory_space=pltpu.SMEM),
    out_shape=jax.ShapeDtypeStruct((1, 1), jnp.float32),
)


def kernel(pos_h, pos_t, pos_r, neg_h, neg_t, neg_r,
           ent_embeddings, rel_embeddings, ent_transfer, rel_transfer):
    gathered = _gather12(ent_embeddings, ent_transfer, rel_embeddings,
                         rel_transfer, pos_h, pos_t, pos_r,
                         neg_h, neg_t, neg_r)
    loss = _loss_call(*gathered)
    return loss[0, 0]


# speedup vs baseline: 2.4696x; 1.3909x over previous
import functools

import jax
import jax.numpy as jnp
from jax import lax
from jax.experimental import pallas as pl
from jax.experimental.pallas import tpu as pltpu
from jax.experimental.pallas import tpu_sc as plsc

BATCH = 16384
DIM = 128
NUM_WORKERS = 32
ROWS_PER_WORKER = BATCH // NUM_WORKERS
CHUNK = 128
NUM_CHUNKS = ROWS_PER_WORKER // CHUNK
MARGIN = 1.0

_ROW_TYPE = jax.ShapeDtypeStruct((BATCH, DIM), jnp.float32)


_NBUF = 4


def _make_gather_kernel():
    mesh = plsc.VectorSubcoreMesh(core_axis_name="c", subcore_axis_name="s")

    @functools.partial(
        pl.kernel,
        out_type=[_ROW_TYPE] * 12,
        mesh=mesh,
        scratch_types=(
            [pltpu.VMEM((6 * NUM_CHUNKS, CHUNK), jnp.int32)]
            + [pltpu.VMEM((CHUNK, DIM), jnp.float32)] * _NBUF
            + [pltpu.SemaphoreType.DMA] * (2 * _NBUF + 1)
        ),
    )
    def gather12(ent_e, ent_t, rel_e, rel_t,
                 ph, pt, pr, nh, nt, nr,
                 o_phe, o_pte, o_php, o_ptp, o_pre, o_prp,
                 o_nhe, o_nte, o_nhp, o_ntp, o_nre, o_nrp,
                 idx_all, r0, r1, r2, r3,
                 g0, g1, g2, g3, w0, w1, w2, w3, isem):
        rows = [r0, r1, r2, r3]
        gsem = [g0, g1, g2, g3]
        wsem = [w0, w1, w2, w3]
        wid = lax.axis_index("s") * 2 + lax.axis_index("c")
        base0 = wid * ROWS_PER_WORKER

        idx_arrays = [ph, pt, pr, nh, nt, nr]
        loads = []
        for a, idx in enumerate(idx_arrays):
            for c in range(NUM_CHUNKS):
                loads.append(pltpu.async_copy(
                    idx.at[pl.ds(base0 + c * CHUNK, CHUNK)],
                    idx_all.at[a * NUM_CHUNKS + c], isem))
        for h in loads:
            h.wait()

        combos = [
            (ent_e, 0, o_phe), (ent_e, 1, o_pte),
            (ent_t, 0, o_php), (ent_t, 1, o_ptp),
            (rel_e, 2, o_pre), (rel_t, 2, o_prp),
            (ent_e, 3, o_nhe), (ent_e, 4, o_nte),
            (ent_t, 3, o_nhp), (ent_t, 4, o_ntp),
            (rel_e, 5, o_nre), (rel_t, 5, o_nrp),
        ]
        steps = [(tbl, a * NUM_CHUNKS + c, out, c)
                 for (tbl, a, out) in combos for c in range(NUM_CHUNKS)]
        n = len(steps)
        hg = [None] * n
        hw = [None] * n
        for k, (tbl, slot, out, c) in enumerate(steps):
            b = k % _NBUF
            if k >= _NBUF:
                hw[k - _NBUF].wait()
            hg[k] = pltpu.async_copy(tbl.at[idx_all.at[slot]], rows[b],
                                     gsem[b])
            if k >= 1:
                j = k - 1
                hg[j].wait()
                _, _, outj, cj = steps[j]
                hw[j] = pltpu.async_copy(
                    rows[j % _NBUF],
                    outj.at[pl.ds(base0 + cj * CHUNK, CHUNK)],
                    wsem[j % _NBUF])
        hg[n - 1].wait()
        _, _, outl, cl = steps[n - 1]
        hw[n - 1] = pltpu.async_copy(
            rows[(n - 1) % _NBUF],
            outl.at[pl.ds(base0 + cl * CHUNK, CHUNK)],
            wsem[(n - 1) % _NBUF])
        for j in range(n - _NBUF, n):
            hw[j].wait()

    return gather12


_gather12 = _make_gather_kernel()

_TC_ROWS = 1024
_TC_GRID = BATCH // _TC_ROWS


def _loss_body(phe, pte, php, ptp, pre, prp,
               nhe, nte, nhp, ntp, nre, nrp, out_ref):
    i = pl.program_id(0)
    he = phe[...]
    te = pte[...]
    sp = jnp.sum(he * php[...] - te * ptp[...], axis=1, keepdims=True)
    dp = he - te + pre[...] + sp * prp[...]
    ps = jnp.sum(dp * dp, axis=1)

    he_n = nhe[...]
    te_n = nte[...]
    sn = jnp.sum(he_n * nhp[...] - te_n * ntp[...], axis=1, keepdims=True)
    dn = he_n - te_n + nre[...] + sn * nrp[...]
    ns = jnp.sum(dn * dn, axis=1)

    partial = jnp.sum(jnp.maximum(ns - ps + MARGIN, 0.0))

    @pl.when(i == 0)
    def _():
        out_ref[0, 0] = 0.0

    out_ref[0, 0] += partial


_row_spec = pl.BlockSpec((_TC_ROWS, DIM), lambda i: (i, 0))

_loss_call = pl.pallas_call(
    _loss_body,
    grid=(_TC_GRID,),
    in_specs=[_row_spec] * 12,
    out_specs=pl.BlockSpec((1, 1), lambda i: (0, 0), memory_space=pltpu.SMEM),
    out_shape=jax.ShapeDtypeStruct((1, 1), jnp.float32),
)


def kernel(pos_h, pos_t, pos_r, neg_h, neg_t, neg_r,
           ent_embeddings, rel_embeddings, ent_transfer, rel_transfer):
    gathered = _gather12(ent_embeddings, ent_transfer, rel_embeddings,
                         rel_transfer, pos_h, pos_t, pos_r,
                         neg_h, neg_t, neg_r)
    loss = _loss_call(*gathered)
    return loss[0, 0]
